# Initial kernel scaffold; baseline (speedup 1.0000x reference)
import jax
import jax.numpy as jnp
from jax.experimental import pallas as pl


def kernel(prev, raw, edge_index, W):
    raise NotImplementedError("write your pallas kernel here")


# speedup vs baseline: 5.2783x; 5.2783x over previous
import functools

import jax
import jax.numpy as jnp
from jax import lax
from jax.experimental import pallas as pl
from jax.experimental.pallas import tpu as pltpu
from jax.experimental.pallas import tpu_sc as plsc

N = 10000
E = 320000
D = 128

NC = 2
NS = 16
NW = NC * NS
L = 16

N_PAD = 10240
CHUNK = 128
EPT = 10112
E_PAD = EPT * NW
N_ROWS16 = N_PAD // L
ROW_BLK = 128

_mesh = plsc.VectorSubcoreMesh(
    core_axis_name="c", subcore_axis_name="s", num_cores=NC, num_subcores=NS)


@functools.partial(
    pl.kernel,
    out_type=jax.ShapeDtypeStruct((NW, N_PAD), jnp.float32),
    mesh=_mesh,
    scratch_types=[
        pltpu.VMEM((EPT,), jnp.int32),
        pltpu.VMEM((N_PAD,), jnp.float32),
    ],
    compiler_params=pltpu.CompilerParams(needs_layout_passes=False),
)
def _deg_sc(dst_hbm, zeros_hbm, out_hbm, dst_v, tab_v):
    c = lax.axis_index("c")
    s = lax.axis_index("s")
    wid = c * NS + s

    pltpu.sync_copy(zeros_hbm, tab_v)
    pltpu.sync_copy(dst_hbm.at[pl.ds(wid * EPT, EPT)], dst_v)

    ones = jnp.full((L,), 1.0, jnp.float32)

    def body(i, _):
        d = dst_v[pl.ds(i * L, L)]
        plsc.addupdate_scatter(tab_v, [d], ones)
        return _

    lax.fori_loop(0, EPT // L, body, None)

    pltpu.sync_copy(tab_v, out_hbm.at[wid])


@functools.partial(
    pl.kernel,
    out_type=jax.ShapeDtypeStruct((NC * N_PAD, D), jnp.float32),
    mesh=_mesh,
    scratch_types=[
        pltpu.VMEM((CHUNK,), jnp.int32),
        pltpu.VMEM((CHUNK,), jnp.int32),
        pltpu.VMEM((CHUNK, D), jnp.float32),
        pltpu.VMEM_SHARED((N_PAD, D), jnp.float32),
        pltpu.SemaphoreType.DMA,
    ],
    compiler_params=pltpu.CompilerParams(needs_layout_passes=False),
)
def _agg_sc(y_hbm, src_hbm, dst_hbm, zeros_hbm, out_hbm, src_v, dst_v,
            rows_v, tab_sh, sem):
    c = lax.axis_index("c")
    s = lax.axis_index("s")
    wid = c * NS + s
    rows = N_PAD // NS

    pltpu.sync_copy(zeros_hbm.at[pl.ds(s * rows, rows)],
                    tab_sh.at[pl.ds(s * rows, rows)])
    plsc.subcore_barrier()

    def body(g, _):
        base = wid * EPT + g * CHUNK
        pltpu.sync_copy(src_hbm.at[pl.ds(base, CHUNK)], src_v)
        pltpu.async_copy(y_hbm.at[src_v], rows_v, sem).wait()
        pltpu.sync_copy(dst_hbm.at[pl.ds(base, CHUNK)], dst_v)
        pltpu.sync_copy(rows_v, tab_sh.at[dst_v], add=True)
        return _

    lax.fori_loop(0, EPT // CHUNK, body, None)

    plsc.subcore_barrier()
    pltpu.sync_copy(tab_sh.at[pl.ds(s * rows, rows)],
                    out_hbm.at[pl.ds(c * N_PAD + s * rows, rows)])


_RB = 1024


def _tca_body(prev_ref, raw_ref, w_ref, u_ref):
    u_ref[...] = jnp.dot(prev_ref[...], w_ref[...],
                         preferred_element_type=jnp.float32) + raw_ref[...]


def _tc_a(prev_p, raw_p, W):
    return pl.pallas_call(
        _tca_body,
        grid=(N_PAD // _RB,),
        in_specs=[
            pl.BlockSpec((_RB, D), lambda i: (i, 0)),
            pl.BlockSpec((_RB, D), lambda i: (i, 0)),
            pl.BlockSpec((D, D), lambda i: (0, 0)),
        ],
        out_specs=pl.BlockSpec((_RB, D), lambda i: (i, 0)),
        out_shape=jax.ShapeDtypeStruct((N_PAD, D), jnp.float32),
    )(prev_p, raw_p, W)


def _norm_from(deg_ref):
    d = jnp.sum(deg_ref[...], axis=0)
    return lax.rsqrt(jnp.maximum(d, 1.0))


def _tcb_body(u_ref, deg_ref, y_ref):
    y_ref[...] = u_ref[...] * _norm_from(deg_ref)


def _tc_b(U, deg3):
    return pl.pallas_call(
        _tcb_body,
        grid=(N_PAD // _RB,),
        in_specs=[
            pl.BlockSpec((_RB, D), lambda i: (i, 0)),
            pl.BlockSpec((NW, _RB, 1), lambda i: (0, i, 0)),
        ],
        out_specs=pl.BlockSpec((_RB, D), lambda i: (i, 0)),
        out_shape=jax.ShapeDtypeStruct((N_PAD, D), jnp.float32),
    )(U, deg3)


def _tcc_body(part_ref, deg_ref, out_ref):
    acc = part_ref[0] + part_ref[1]
    out_ref[...] = jnp.maximum(acc * _norm_from(deg_ref), 0.0)


def _tc_c(part, deg3):
    return pl.pallas_call(
        _tcc_body,
        grid=(N_PAD // _RB,),
        in_specs=[
            pl.BlockSpec((NC, _RB, D), lambda i: (0, i, 0)),
            pl.BlockSpec((NW, _RB, 1), lambda i: (0, i, 0)),
        ],
        out_specs=pl.BlockSpec((_RB, D), lambda i: (i, 0)),
        out_shape=jax.ShapeDtypeStruct((N_PAD, D), jnp.float32),
    )(part, deg3)


def kernel(prev, raw, edge_index, W):
    src = edge_index[0]
    dst = edge_index[1]
    src_p = jnp.pad(src, (0, E_PAD - E))
    dst_p = jnp.pad(dst, (0, E_PAD - E), constant_values=N)
    prev_p = jnp.pad(prev, ((0, N_PAD - N), (0, 0)))
    raw_p = jnp.pad(raw, ((0, N_PAD - N), (0, 0)))

    zeros1 = jnp.zeros((N_PAD,), jnp.float32)
    zeros_d = jnp.zeros((N_PAD, D), jnp.float32)

    deg2 = _deg_sc(dst_p, zeros1)
    U = _tc_a(prev_p, raw_p, W)
    deg3 = deg2.reshape(NW, N_PAD, 1)
    Y = _tc_b(U, deg3)
    part = _agg_sc(Y, src_p, dst_p, zeros_d)
    out = _tc_c(part.reshape(NC, N_PAD, D), deg3)
    return out[:N]
